# baseline (speedup 1.0000x reference)
import functools

import jax
import jax.numpy as jnp
from jax.experimental import pallas as pl
from jax.experimental.pallas import tpu as pltpu


def _diag_loss_kernel(x_hbm, out_ref, buf, sems, *, batch, scale):
    def slice_copy(b):
        return pltpu.make_async_copy(x_hbm.at[b, b], buf.at[b], sems.at[b])

    for b in range(batch):
        slice_copy(b).start()
    for b in range(batch):
        slice_copy(b).wait()

    out_ref[0, 0] = jnp.sum(buf[...].astype(jnp.float32)) * jnp.float32(scale)


def kernel(output):
    B, C, H, W = output.shape
    scale = -1.0 / float(H * W)

    loss = pl.pallas_call(
        functools.partial(_diag_loss_kernel, batch=B, scale=scale),
        out_shape=jax.ShapeDtypeStruct((1, 1), jnp.float32),
        in_specs=[pl.BlockSpec(memory_space=pl.ANY)],
        out_specs=pl.BlockSpec(memory_space=pltpu.SMEM),
        scratch_shapes=[
            pltpu.VMEM((B, H, W), output.dtype),
            pltpu.SemaphoreType.DMA((B,)),
        ],
    )(output)
    return loss[0, 0]


# speedup vs baseline: 7.0386x; 7.0386x over previous
import functools

import jax
import jax.numpy as jnp
from jax.experimental import pallas as pl
from jax.experimental.pallas import tpu as pltpu


def _diag_loss_kernel(x_ref, out_ref, *, scale, nsteps):
    b = pl.program_id(0)
    blk = x_ref[0].astype(jnp.float32)
    s_wc = jnp.sum(blk, axis=0)
    s_c = jnp.sum(s_wc, axis=0, keepdims=True)
    lane = jax.lax.broadcasted_iota(jnp.int32, s_c.shape, 1)
    part = jnp.sum(jnp.where(lane == b, s_c, 0.0)) * jnp.float32(scale)

    @pl.when(b == 0)
    def _():
        out_ref[0, 0] = part

    @pl.when(b != 0)
    def _():
        out_ref[0, 0] += part


def kernel(output):
    B, C, H, W = output.shape
    scale = -1.0 / float(H * W)

    x = jnp.transpose(output, (0, 2, 3, 1))

    cw = min(C, max(128, -(-B // 128) * 128))

    loss = pl.pallas_call(
        functools.partial(_diag_loss_kernel, scale=scale, nsteps=B),
        out_shape=jax.ShapeDtypeStruct((1, 1), jnp.float32),
        grid=(B,),
        in_specs=[pl.BlockSpec((1, H, W, cw), lambda b: (b, 0, 0, 0))],
        out_specs=pl.BlockSpec((1, 1), lambda b: (0, 0),
                               memory_space=pltpu.SMEM),
        compiler_params=pltpu.CompilerParams(
            dimension_semantics=("arbitrary",)),
    )(x)
    return loss[0, 0]
